# baseline (speedup 1.0000x reference)
import functools

import jax
import jax.numpy as jnp
from jax import lax
from jax.experimental import pallas as pl
from jax.experimental.pallas import tpu as pltpu
from jax.experimental.pallas import tpu_sc as plsc

DIM = 32
HIST = 200
BATCH = 16384
NUM_CORES = 2
NUM_SUBCORES = 16
NUM_WORKERS = NUM_CORES * NUM_SUBCORES
TT = HIST // 8
KT = BATCH // 128
K_PER_W = KT // NUM_WORKERS
N_UNITS = TT * K_PER_W
N_HALF = 2 * N_UNITS
LPAD = 129
NB = 3


@jax.jit
def _sc_embedding_gather(x4, table):
    mesh = plsc.VectorSubcoreMesh(core_axis_name="c", subcore_axis_name="s")

    @functools.partial(
        pl.kernel,
        mesh=mesh,
        out_type=jax.ShapeDtypeStruct((HIST, DIM // 8, KT, 8, 128), jnp.float32),
        scratch_types=[
            pltpu.VMEM((NB, 1024), jnp.int32),
            pltpu.VMEM((NB, 4, 128, DIM), jnp.float32),
            pltpu.VMEM((NB, 4, DIM // 8, 8, LPAD), jnp.float32),
        ]
        + [pltpu.SemaphoreType.DMA] * (3 * NB),
        compiler_params=pltpu.CompilerParams(
            use_tc_tiling_on_sc=False, needs_layout_passes=False),
    )
    def k(x4_hbm, table_hbm, out_hbm, idx_v, rows_v, trans_v, *sems):
        isem = sems[0:NB]
        gsem = sems[NB:2 * NB]
        ssem = sems[2 * NB:3 * NB]
        wid = lax.axis_index("s") * NUM_CORES + lax.axis_index("c")
        iota = lax.iota(jnp.int32, 16)
        jv = [iota >> 3, 2 + (iota >> 3)]
        sv = iota & 7

        def load_idx(u, bu):
            tt = u // K_PER_W
            kk = wid * K_PER_W + (u % K_PER_W)
            pltpu.async_copy(x4_hbm.at[tt, kk], idx_v.at[bu], isem[bu])

        def wait_idx(bu):
            pltpu.make_async_copy(x4_hbm.at[0, 0], idx_v.at[bu], isem[bu]).wait()

        def fire_gathers(h, q, bu):
            for ts in range(4):
                sl = pl.ds((4 * h + ts) * 128, 128)
                pltpu.async_copy(
                    table_hbm.at[idx_v.at[bu, sl]], rows_v.at[q, ts], gsem[q])

        def wait_gathers(q):
            for ts in range(4):
                pltpu.make_async_copy(
                    table_hbm.at[idx_v.at[0, pl.ds(0, 128)]],
                    rows_v.at[q, ts], gsem[q]).wait()

        def transpose(q):
            @plsc.parallel_loop(0, 128, unroll=4)
            def _(l):
                lv = jnp.full((16,), l, jnp.int32)
                for ts in range(4):
                    for di in range(2):
                        vals = rows_v[q, ts, l, pl.ds(16 * di, 16)]
                        plsc.store_scatter(
                            trans_v.at[q, ts], [jv[di], sv, lv], vals)

        def fire_stores(u, h, q):
            tt = u // K_PER_W
            kk = wid * K_PER_W + (u % K_PER_W)
            for ts in range(4):
                t = tt * 8 + 4 * h + ts
                pltpu.async_copy(
                    trans_v.at[q, ts, :, :, pl.ds(0, 128)],
                    out_hbm.at[t, :, kk], ssem[q])

        def wait_stores(q):
            for ts in range(4):
                pltpu.make_async_copy(
                    trans_v.at[q, ts, :, :, pl.ds(0, 128)],
                    out_hbm.at[0, :, 0], ssem[q]).wait()

        def step(H, p):
            q = p % NB
            h = p % 2
            u = H // 2
            wait_gathers(q)
            wait_stores(q)
            transpose(q)
            fire_stores(u, h, q)
            h3 = (p + 1) % 2
            iu3 = ((p + 3) // 2) % NB

            @pl.when(H < N_HALF - NB)
            def _():
                if h3 == 0:
                    wait_idx(iu3)
                fire_gathers(h3, q, iu3)

            if h == 1:
                @pl.when(u + NB < N_UNITS)
                def _():
                    load_idx(u + NB, p // 2)

        for bu in range(NB):
            load_idx(bu, bu)
        wait_idx(0)
        fire_gathers(0, 0, 0)
        fire_gathers(1, 1, 0)
        wait_idx(1)
        fire_gathers(0, 2, 1)
        fire_stores(0, 0, 0)
        fire_stores(0, 1, 1)
        fire_stores(1, 0, 2)

        def body(g, carry):
            H0 = 6 * g
            for p in range(6):
                step(H0 + p, p)
            return carry

        lax.fori_loop(0, (N_HALF - 2) // 6, body, 0)

        for H in (N_HALF - 2, N_HALF - 1):
            p = H % 6
            q = p % NB
            wait_gathers(q)
            wait_stores(q)
            transpose(q)
            fire_stores(H // 2, H % 2, q)
        for q in range(NB):
            wait_stores(q)

    return k(x4, table)


def kernel(x, table):
    x4 = (
        x.astype(jnp.int32).T
        .reshape(TT, 8, KT, 128)
        .transpose(0, 2, 1, 3)
        .reshape(TT, KT, 1024)
    )
    out5 = _sc_embedding_gather(x4, table)
    return out5.transpose(2, 4, 0, 1, 3).reshape(BATCH, HIST, DIM)


# speedup vs baseline: 1.0040x; 1.0040x over previous
import functools

import jax
import jax.numpy as jnp
from jax import lax
from jax.experimental import pallas as pl
from jax.experimental.pallas import tpu as pltpu
from jax.experimental.pallas import tpu_sc as plsc

DIM = 32
HIST = 200
BATCH = 16384
NUM_CORES = 2
NUM_SUBCORES = 16
NUM_WORKERS = NUM_CORES * NUM_SUBCORES
TT = HIST // 8
KT = BATCH // 128
K_PER_W = KT // NUM_WORKERS
N_UNITS = TT * K_PER_W
N_HALF = 2 * N_UNITS
LPAD = 129
NB = 3


@jax.jit
def _sc_embedding_gather(x4, table):
    mesh = plsc.VectorSubcoreMesh(core_axis_name="c", subcore_axis_name="s")

    @functools.partial(
        pl.kernel,
        mesh=mesh,
        out_type=jax.ShapeDtypeStruct((HIST, DIM // 8, KT, 8, 128), jnp.float32),
        scratch_types=[
            pltpu.VMEM((NB, 1024), jnp.int32),
            pltpu.VMEM((NB, 4, 128, DIM), jnp.float32),
            pltpu.VMEM((NB, 4, DIM // 8, 8, LPAD), jnp.float32),
        ]
        + [pltpu.SemaphoreType.DMA] * (3 * NB),
        compiler_params=pltpu.CompilerParams(
            use_tc_tiling_on_sc=False, needs_layout_passes=False),
    )
    def k(x4_hbm, table_hbm, out_hbm, idx_v, rows_v, trans_v, *sems):
        isem = sems[0:NB]
        gsem = sems[NB:2 * NB]
        ssem = sems[2 * NB:3 * NB]
        wid = lax.axis_index("s") * NUM_CORES + lax.axis_index("c")
        iota = lax.iota(jnp.int32, 16)
        jv = [iota >> 3, 2 + (iota >> 3)]
        sv = iota & 7

        def load_idx(u, bu):
            tt = u // K_PER_W
            kk = wid * K_PER_W + (u % K_PER_W)
            pltpu.async_copy(x4_hbm.at[tt, kk], idx_v.at[bu], isem[bu])

        def wait_idx(bu):
            pltpu.make_async_copy(x4_hbm.at[0, 0], idx_v.at[bu], isem[bu]).wait()

        def fire_gathers(h, q, bu):
            for ts in range(4):
                sl = pl.ds((4 * h + ts) * 128, 128)
                pltpu.async_copy(
                    table_hbm.at[idx_v.at[bu, sl]], rows_v.at[q, ts], gsem[q])

        def wait_gathers(q):
            for ts in range(4):
                pltpu.make_async_copy(
                    table_hbm.at[idx_v.at[0, pl.ds(0, 128)]],
                    rows_v.at[q, ts], gsem[q]).wait()

        def transpose(q):
            @plsc.parallel_loop(0, 128, unroll=2)
            def _(l):
                lv = jnp.full((16,), l, jnp.int32)
                for ts in range(4):
                    for di in range(2):
                        vals = rows_v[q, ts, l, pl.ds(16 * di, 16)]
                        plsc.store_scatter(
                            trans_v.at[q, ts], [jv[di], sv, lv], vals)

        def fire_stores(u, h, q):
            tt = u // K_PER_W
            kk = wid * K_PER_W + (u % K_PER_W)
            for ts in range(4):
                t = tt * 8 + 4 * h + ts
                pltpu.async_copy(
                    trans_v.at[q, ts, :, :, pl.ds(0, 128)],
                    out_hbm.at[t, :, kk], ssem[q])

        def wait_stores(q):
            for ts in range(4):
                pltpu.make_async_copy(
                    trans_v.at[q, ts, :, :, pl.ds(0, 128)],
                    out_hbm.at[0, :, 0], ssem[q]).wait()

        def step(H, p):
            q = p % NB
            h = p % 2
            u = H // 2
            wait_gathers(q)
            wait_stores(q)
            transpose(q)
            fire_stores(u, h, q)
            h3 = (p + 1) % 2
            iu3 = ((p + 3) // 2) % NB

            @pl.when(H < N_HALF - NB)
            def _():
                if h3 == 0:
                    wait_idx(iu3)
                fire_gathers(h3, q, iu3)

            if h == 1:
                @pl.when(u + NB < N_UNITS)
                def _():
                    load_idx(u + NB, p // 2)

        for bu in range(NB):
            load_idx(bu, bu)
        wait_idx(0)
        fire_gathers(0, 0, 0)
        fire_gathers(1, 1, 0)
        wait_idx(1)
        fire_gathers(0, 2, 1)
        fire_stores(0, 0, 0)
        fire_stores(0, 1, 1)
        fire_stores(1, 0, 2)

        def body(g, carry):
            H0 = 6 * g
            for p in range(6):
                step(H0 + p, p)
            return carry

        lax.fori_loop(0, (N_HALF - 2) // 6, body, 0)

        for H in (N_HALF - 2, N_HALF - 1):
            p = H % 6
            q = p % NB
            wait_gathers(q)
            wait_stores(q)
            transpose(q)
            fire_stores(H // 2, H % 2, q)
        for q in range(NB):
            wait_stores(q)

    return k(x4, table)


def kernel(x, table):
    x4 = (
        x.astype(jnp.int32).T
        .reshape(TT, 8, KT, 128)
        .transpose(0, 2, 1, 3)
        .reshape(TT, KT, 1024)
    )
    out5 = _sc_embedding_gather(x4, table)
    return out5.transpose(2, 4, 0, 1, 3).reshape(BATCH, HIST, DIM)
